# baseline (speedup 1.0000x reference)
import functools

import jax
import jax.numpy as jnp
from jax.experimental import pallas as pl
from jax.experimental.pallas import tpu as pltpu


def _kmeans_loss_body(x_ref, mu_ref, o_ref, lhs_ref, *, inv_g):
    bf = jnp.bfloat16

    @pl.when(pl.program_id(0) == 0)
    def _build_lhs():
        m = mu_ref[...]
        musq = jnp.sum(m * m, axis=0, keepdims=True)
        ones_row = jnp.ones((1, m.shape[1]), bf)
        lhs_ref[...] = jnp.concatenate(
            [m.astype(bf), (-0.5 * musq).astype(bf), ones_row], axis=0)

    x = x_ref[...]
    xsq = jnp.sum(x * x, axis=1, keepdims=True)
    ones_col = jnp.ones((x.shape[0], 1), bf)
    rhs = jnp.concatenate(
        [x.astype(bf), ones_col, (-0.5 * xsq).astype(bf)], axis=1)
    p = jax.lax.dot_general(
        lhs_ref[...], rhs, (((0,), (1,)), ((), ())),
        preferred_element_type=jnp.float32)
    pmax = jnp.max(p, axis=0, keepdims=True)
    o_ref[...] = pmax * (-2.0 * inv_g)


def kernel(images, mu):
    B, G = images.shape
    _, K = mu.shape
    nb = 4
    bb = B // nb
    out = pl.pallas_call(
        functools.partial(_kmeans_loss_body, inv_g=1.0 / G),
        out_shape=jax.ShapeDtypeStruct((1, B), jnp.float32),
        grid=(nb,),
        in_specs=[
            pl.BlockSpec((bb, G), lambda i: (i, 0)),
            pl.BlockSpec((G, K), lambda i: (0, 0)),
        ],
        out_specs=pl.BlockSpec((1, bb), lambda i: (0, i)),
        scratch_shapes=[pltpu.VMEM((G + 2, K), jnp.bfloat16)],
        compiler_params=pltpu.CompilerParams(
            dimension_semantics=("arbitrary",),
        ),
    )(images, mu)
    return out.reshape(B)


# speedup vs baseline: 1.0583x; 1.0583x over previous
import functools

import jax
import jax.numpy as jnp
from jax.experimental import pallas as pl
from jax.experimental.pallas import tpu as pltpu

_NCHUNK = 4


def _kmeans_loss_body(x_hbm, mu_hbm, o_ref, lhs_ref, rhs_ref, sem_x, sem_m,
                      *, inv_g):
    B, G = x_hbm.shape
    K = mu_hbm.shape[1]
    kc = K // _NCHUNK

    cp_x = pltpu.make_async_copy(x_hbm, rhs_ref.at[:, 0:G], sem_x)
    cp_x.start()
    cp_m = [
        pltpu.make_async_copy(mu_hbm.at[:, j * kc:(j + 1) * kc],
                              lhs_ref.at[0:G, j * kc:(j + 1) * kc],
                              sem_m.at[j])
        for j in range(_NCHUNK)
    ]
    for cp in cp_m:
        cp.start()

    cp_x.wait()
    x = rhs_ref[:, 0:G]
    xsq = jnp.sum(x * x, axis=1, keepdims=True)
    rhs_ref[:, G:G + 1] = jnp.ones((B, 1), jnp.float32)
    rhs_ref[:, G + 1:G + 2] = -0.5 * xsq

    run = None
    for j in range(_NCHUNK):
        cp_m[j].wait()
        js = slice(j * kc, (j + 1) * kc)
        mj = lhs_ref[0:G, js]
        musq = jnp.sum(mj * mj, axis=0, keepdims=True)
        lhs_ref[G:G + 1, js] = -0.5 * musq
        lhs_ref[G + 1:G + 2, js] = jnp.ones_like(musq)
        p = jax.lax.dot_general(
            lhs_ref[:, js], rhs_ref[...], (((0,), (1,)), ((), ())),
            preferred_element_type=jnp.float32)
        pmax = jnp.max(p, axis=0, keepdims=True)
        run = pmax if run is None else jnp.maximum(run, pmax)
    o_ref[...] = run * (-2.0 * inv_g)


def kernel(images, mu):
    B, G = images.shape
    _, K = mu.shape
    out = pl.pallas_call(
        functools.partial(_kmeans_loss_body, inv_g=1.0 / G),
        out_shape=jax.ShapeDtypeStruct((1, B), jnp.float32),
        in_specs=[
            pl.BlockSpec(memory_space=pl.ANY),
            pl.BlockSpec(memory_space=pl.ANY),
        ],
        out_specs=pl.BlockSpec((1, B), lambda: (0, 0)),
        scratch_shapes=[
            pltpu.VMEM((G + 2, K), jnp.float32),
            pltpu.VMEM((B, G + 2), jnp.float32),
            pltpu.SemaphoreType.DMA,
            pltpu.SemaphoreType.DMA((_NCHUNK,)),
        ],
    )(images, mu)
    return out.reshape(B)


# speedup vs baseline: 1.3205x; 1.2478x over previous
import functools

import jax
import jax.numpy as jnp
from jax.experimental import pallas as pl


def _kmeans_loss_body(x_ref, mu_ref, o_ref, *, inv_g):
    x = x_ref[...]
    m = mu_ref[...]
    b = x.shape[0]
    bf = jnp.bfloat16
    musq = jnp.sum(m * m, axis=0, keepdims=True)
    ones_row = jnp.ones((1, m.shape[1]), bf)
    lhs = jnp.concatenate(
        [m.astype(bf), (-0.5 * musq).astype(bf), ones_row], axis=0)
    xsq = jnp.sum(x * x, axis=1, keepdims=True)
    ones_col = jnp.ones((b, 1), bf)
    rhs = jnp.concatenate(
        [x.astype(bf), ones_col, (-0.5 * xsq).astype(bf)], axis=1)
    p = jax.lax.dot_general(
        lhs, rhs, (((0,), (1,)), ((), ())),
        preferred_element_type=jnp.float32)
    pmax = jnp.max(p, axis=0, keepdims=True)
    o_ref[...] = pmax * (-2.0 * inv_g)


def kernel(images, mu):
    B, G = images.shape
    _, K = mu.shape
    out = pl.pallas_call(
        functools.partial(_kmeans_loss_body, inv_g=1.0 / G),
        out_shape=jax.ShapeDtypeStruct((1, B), jnp.float32),
        grid=(1,),
        in_specs=[
            pl.BlockSpec((B, G), lambda i: (0, 0)),
            pl.BlockSpec((G, K), lambda i: (0, 0)),
        ],
        out_specs=pl.BlockSpec((1, B), lambda i: (0, 0)),
    )(images, mu)
    return out.reshape(B)
